# baseline (speedup 1.0000x reference)
import functools

import jax
import jax.numpy as jnp
from jax import lax
from jax.experimental import pallas as pl
from jax.experimental.pallas import tpu as pltpu
from jax.experimental.pallas import tpu_sc as plsc

_B = 16384
_SUB = 32
_TROWS = 54
_NUM_WORKERS = 32
_CHUNK = _B // _NUM_WORKERS
_GROUPS = _CHUNK // 16
_OUT_PER_WORKER = 4 * _CHUNK * _SUB


def _body(ts_hbm, table_hbm, out_hbm, ts_v, table_v, dst_v, sem):
    wid = lax.axis_index("s") * 2 + lax.axis_index("c")
    base = wid * _CHUNK
    pltpu.async_copy(table_hbm, table_v, sem)
    pltpu.async_copy(ts_hbm.at[pl.ds(base, _CHUNK)], ts_v, sem).wait()
    pltpu.make_async_copy(table_hbm, table_v, sem).wait()

    lanes = lax.iota(jnp.int32, 16)
    obase = wid * _OUT_PER_WORKER

    def group_step(g, carry):
        ts = ts_v[pl.ds(g * 16, 16)]
        dn0 = (ts.astype(jnp.float32) * jnp.float32(1.0 / 86400.0)).astype(
            jnp.int32
        )
        r = ts - dn0 * 86400
        dn = dn0 + jnp.where(r >= 86400, 1, 0) - jnp.where(r < 0, 1, 0)
        dow = dn - ((dn * 18725) >> 17) * 7
        doy = dn - ((dn * 22983) >> 23) * 365
        q30 = (doy * 1093) >> 15
        month = q30 - jnp.where(q30 >= 12, 12, 0)
        dom = doy - ((doy * 4229) >> 17) * 31
        quarter = (month * 11) >> 5

        addr = [
            dow * _SUB,
            (month + 7) * _SUB,
            (dom + 19) * _SUB,
            (quarter + 50) * _SUB,
        ]
        gbase = 64 * _SUB * g

        for b0 in range(0, 16, 2):
            vals = []
            for b in (b0, b0 + 1):
                for k in range(4):
                    rowoff = lax.gather(
                        addr[k],
                        jnp.full((16, 1), b, jnp.int32),
                        lax.GatherDimensionNumbers(
                            offset_dims=(),
                            collapsed_slice_dims=(0,),
                            start_index_map=(0,),
                        ),
                        (1,),
                        mode=lax.GatherScatterMode.PROMISE_IN_BOUNDS,
                    )
                    for m in (0, 16):
                        vals.append(
                            plsc.load_gather(table_v, [rowoff + (m + lanes)])
                        )
            i = 0
            for b in (b0, b0 + 1):
                dbase = gbase + 128 * b
                for k in range(4):
                    for m in (0, 16):
                        dst_v[pl.ds(dbase + 32 * k + m, 16)] = vals[i]
                        i += 1
        @pl.when(lax.rem(g, 4) == 3)
        def _():
            sbase = gbase - 192 * _SUB
            pltpu.async_copy(
                dst_v.at[pl.ds(sbase, 256 * _SUB)],
                out_hbm.at[pl.ds(obase + sbase, 256 * _SUB)],
                sem,
            )
        return carry

    lax.fori_loop(0, _GROUPS, group_step, 0)

    for _ in range(_GROUPS // 4):
        pltpu.make_async_copy(
            dst_v.at[pl.ds(0, 256 * _SUB)],
            out_hbm.at[pl.ds(obase, 256 * _SUB)],
            sem,
        ).wait()


@functools.partial(jax.jit)
def _sc_lookup(ts, table_flat):
    mesh = plsc.VectorSubcoreMesh(core_axis_name="c", subcore_axis_name="s")
    k = functools.partial(
        pl.kernel,
        mesh=mesh,
        out_type=jax.ShapeDtypeStruct((4 * _B * _SUB,), jnp.float32),
        scratch_types=[
            pltpu.VMEM((_CHUNK,), jnp.int32),
            pltpu.VMEM((_TROWS * _SUB,), jnp.float32),
            pltpu.VMEM((_OUT_PER_WORKER,), jnp.float32),
            pltpu.SemaphoreType.DMA,
        ],
        compiler_params=pltpu.CompilerParams(
            use_tc_tiling_on_sc=False,
            needs_layout_passes=False,
        ),
    )(_body)
    return k(ts, table_flat)


def kernel(timestamps, dow_table, month_table, dom_table, quarter_table):
    table = jnp.concatenate(
        [dow_table, month_table, dom_table, quarter_table], axis=0
    ).reshape(-1)
    ts = timestamps.astype(jnp.int32)
    out = _sc_lookup(ts, table)
    return out.reshape(_B, 4 * _SUB)


# speedup vs baseline: 1.0036x; 1.0036x over previous
import functools

import jax
import jax.numpy as jnp
from jax import lax
from jax.experimental import pallas as pl
from jax.experimental.pallas import tpu as pltpu
from jax.experimental.pallas import tpu_sc as plsc

_B = 16384
_SUB = 32
_TROWS = 54
_NUM_WORKERS = 32
_CHUNK = _B // _NUM_WORKERS
_GROUPS = _CHUNK // 16
_OUT_PER_WORKER = 4 * _CHUNK * _SUB


def _body(ts_hbm, table_hbm, out_hbm, ts_v, table_v, dst_v, sem):
    wid = lax.axis_index("s") * 2 + lax.axis_index("c")
    base = wid * _CHUNK
    pltpu.async_copy(table_hbm, table_v, sem)
    pltpu.async_copy(ts_hbm.at[pl.ds(base, _CHUNK)], ts_v, sem).wait()
    pltpu.make_async_copy(table_hbm, table_v, sem).wait()

    lanes = lax.iota(jnp.int32, 16)
    obase = wid * _OUT_PER_WORKER

    def group_step(g, carry):
        ts = ts_v[pl.ds(g * 16, 16)]
        dn0 = (ts.astype(jnp.float32) * jnp.float32(1.0 / 86400.0)).astype(
            jnp.int32
        )
        r = ts - dn0 * 86400
        dn = dn0 + jnp.where(r >= 86400, 1, 0) - jnp.where(r < 0, 1, 0)
        dow = dn - ((dn * 18725) >> 17) * 7
        doy = dn - ((dn * 22983) >> 23) * 365
        q30 = (doy * 1093) >> 15
        month = q30 - jnp.where(q30 >= 12, 12, 0)
        dom = doy - ((doy * 4229) >> 17) * 31
        quarter = (month * 11) >> 5

        addr = [
            dow * _SUB,
            (month + 7) * _SUB,
            (dom + 19) * _SUB,
            (quarter + 50) * _SUB,
        ]
        gbase = 64 * _SUB * g

        for b0 in range(0, 16, 2):
            vals = []
            for b in (b0, b0 + 1):
                for k in range(4):
                    rowoff = lax.gather(
                        addr[k],
                        jnp.full((16, 1), b, jnp.int32),
                        lax.GatherDimensionNumbers(
                            offset_dims=(),
                            collapsed_slice_dims=(0,),
                            start_index_map=(0,),
                        ),
                        (1,),
                        mode=lax.GatherScatterMode.PROMISE_IN_BOUNDS,
                    )
                    for m in (0, 16):
                        vals.append(
                            plsc.load_gather(table_v, [rowoff + (m + lanes)])
                        )
            i = 0
            for b in (b0, b0 + 1):
                dbase = gbase + 128 * b
                for k in range(4):
                    for m in (0, 16):
                        dst_v[pl.ds(dbase + 32 * k + m, 16)] = vals[i]
                        i += 1
        @pl.when(lax.rem(g, 2) == 1)
        def _():
            sbase = gbase - 64 * _SUB
            pltpu.async_copy(
                dst_v.at[pl.ds(sbase, 128 * _SUB)],
                out_hbm.at[pl.ds(obase + sbase, 128 * _SUB)],
                sem,
            )
        return carry

    lax.fori_loop(0, _GROUPS, group_step, 0)

    for _ in range(_GROUPS // 2):
        pltpu.make_async_copy(
            dst_v.at[pl.ds(0, 128 * _SUB)],
            out_hbm.at[pl.ds(obase, 128 * _SUB)],
            sem,
        ).wait()


@functools.partial(jax.jit)
def _sc_lookup(ts, table_flat):
    mesh = plsc.VectorSubcoreMesh(core_axis_name="c", subcore_axis_name="s")
    k = functools.partial(
        pl.kernel,
        mesh=mesh,
        out_type=jax.ShapeDtypeStruct((4 * _B * _SUB,), jnp.float32),
        scratch_types=[
            pltpu.VMEM((_CHUNK,), jnp.int32),
            pltpu.VMEM((_TROWS * _SUB,), jnp.float32),
            pltpu.VMEM((_OUT_PER_WORKER,), jnp.float32),
            pltpu.SemaphoreType.DMA,
        ],
        compiler_params=pltpu.CompilerParams(
            use_tc_tiling_on_sc=False,
            needs_layout_passes=False,
        ),
    )(_body)
    return k(ts, table_flat)


def kernel(timestamps, dow_table, month_table, dom_table, quarter_table):
    table = jnp.concatenate(
        [dow_table, month_table, dom_table, quarter_table], axis=0
    ).reshape(-1)
    ts = timestamps.astype(jnp.int32)
    out = _sc_lookup(ts, table)
    return out.reshape(_B, 4 * _SUB)
